# baseline (speedup 1.0000x reference)
import functools

import jax
import jax.numpy as jnp
from jax import lax
from jax.experimental import pallas as pl
from jax.experimental.pallas import tpu as pltpu
from jax.experimental.pallas import tpu_sc as plsc

DIM = 64
NC = 2
NS = 16
NW = NC * NS
BLK = 512
SCAN_CHUNK = 8192
MCAP = 7056
SCAP = 192


@functools.cache
def _make_kernel(B, V):
    nfull = V // BLK
    tail0 = nfull * BLK
    bpw = -(-nfull // NW)
    nscan = B // SCAN_CHUNK
    mesh = plsc.VectorSubcoreMesh(core_axis_name="c", subcore_axis_name="s")

    @functools.partial(
        pl.kernel,
        mesh=mesh,
        compiler_params=pltpu.CompilerParams(needs_layout_passes=False),
        out_type=jax.ShapeDtypeStruct((B + SCAP, 2 * DIM), jnp.float32),
        scratch_types=[
            pltpu.VMEM((SCAN_CHUNK,), jnp.int32),
            pltpu.VMEM((MCAP,), jnp.int32),
            pltpu.VMEM((MCAP,), jnp.int32),
            pltpu.VMEM((DIM, BLK), jnp.float32),
            pltpu.VMEM((8, DIM), jnp.float32),
            pltpu.VMEM((SCAP,), jnp.int32),
            pltpu.VMEM((SCAP,), jnp.int32),
            pltpu.VMEM((SCAP, 2 * DIM), jnp.float32),
            pltpu.VMEM((16,), jnp.int32),
            pltpu.VMEM((16,), jnp.int32),
            pltpu.SemaphoreType.DMA,
        ],
    )
    def k(idx_hbm, wt_hbm, tail_hbm, out_hbm,
          icv, midx, mg, slab, tailv, lidx, glist, stag, slot_a, slot_b, wsem):
        wid = lax.axis_index("s") * NC + lax.axis_index("c")
        iota16 = lax.iota(jnp.int32, 16)
        slot_ar = slot_a.at[pl.ds(0, 16)]
        slot_br = slot_b.at[pl.ds(0, 16)]
        midx_r = midx.at[pl.ds(0, MCAP)]
        mg_r = mg.at[pl.ds(0, MCAP)]
        lidx_r = lidx.at[pl.ds(0, SCAP)]
        glist_r = glist.at[pl.ds(0, SCAP)]
        slab_r = slab.at[pl.ds(0, DIM), pl.ds(0, BLK)]
        tailv_r = tailv.at[pl.ds(0, 8), pl.ds(0, DIM)]
        stag_r = stag.at[pl.ds(0, SCAP), pl.ds(0, 2 * DIM)]
        wid_v = jnp.zeros((16,), jnp.int32) + wid

        a0 = pl.multiple_of((wid * 2) & ~jnp.int32(7), 8)
        pltpu.sync_copy(tail_hbm.at[pl.ds(a0, 8)], tailv)

        cnt = jnp.int32(0)
        for c in range(nscan):
            pltpu.sync_copy(idx_hbm.at[pl.ds(c * SCAN_CHUNK, SCAN_CHUNK)], icv)

            def scan_body(i, cnt, c=c):
                v = icv[pl.ds(i * 16, 16)]
                own = jnp.where(
                    v >= tail0,
                    lax.shift_right_logical(v - tail0, 1),
                    lax.shift_right_logical(v, 9) & (NW - 1),
                )
                m = own == wid_v
                pcs = lax.reduce_max(plsc.all_reduce_population_count(m), (0,))

                @pl.when(pcs > 0)
                def _():
                    plsc.store_compressed(slot_ar, v, mask=m)
                    gv = iota16 + (c * SCAN_CHUNK + i * 16)
                    plsc.store_compressed(slot_br, gv, mask=m)
                    lanes = iota16 + cnt
                    keep = iota16 < pcs
                    plsc.store_scatter(midx_r, [lanes], slot_a[...], mask=keep)
                    plsc.store_scatter(mg_r, [lanes], slot_b[...], mask=keep)

                return cnt + pcs

            cnt = lax.fori_loop(0, SCAN_CHUNK // 16, scan_body, cnt)

        nvec = lax.shift_right_logical(cnt + 15, 4)
        dump_v = jnp.zeros((16,), jnp.int32) + B

        def rescan(lo_s, hi_s):
            def resc(ii, cnt2):
                mv = midx[pl.ds(ii * 16, 16)]
                gv = mg[pl.ds(ii * 16, 16)]
                valid = (iota16 + ii * 16) < cnt
                m2 = valid & (mv >= lo_s) & (mv < hi_s)
                pc2 = lax.reduce_max(plsc.all_reduce_population_count(m2), (0,))

                @pl.when(pc2 > 0)
                def _():
                    plsc.store_compressed(slot_ar, mv - lo_s, mask=m2)
                    plsc.store_compressed(slot_br, gv, mask=m2)
                    lanes = iota16 + cnt2
                    keep = iota16 < pc2
                    plsc.store_scatter(lidx_r, [lanes], slot_a[...], mask=keep)
                    plsc.store_scatter(glist_r, [lanes], slot_b[...], mask=keep)

                return cnt2 + pc2

            return lax.fori_loop(0, nvec, resc, jnp.int32(0))

        def prefill_glist():
            for kk in range(SCAP // 16):
                glist[pl.ds(kk * 16, 16)] = dump_v

        def block_body(bi, carry):
            bid = bi * NW + wid

            @pl.when(bid < nfull)
            def _():
                c0 = pl.multiple_of(bid * BLK, BLK)
                pltpu.sync_copy(wt_hbm.at[:, pl.ds(c0, BLK)], slab)
                prefill_glist()
                cnt2 = rescan(c0, c0 + BLK)

                def gbody(mm, _):
                    mm_v = jnp.zeros((16,), jnp.int32) + mm
                    r_v = plsc.load_gather(lidx_r, [mm_v])
                    for q in range(DIM // 16):
                        vec = plsc.load_gather(slab_r, [iota16 + q * 16, r_v])
                        plsc.store_scatter(stag_r, [mm_v, iota16 + q * 16], vec)
                    return 0

                lax.fori_loop(0, cnt2, gbody, 0)
                pltpu.async_copy(stag, out_hbm.at[glist], wsem).wait()

            return carry

        lax.fori_loop(0, bpw, block_body, 0)

        prefill_glist()
        lo_s = tail0 + wid * 2
        cnt3 = rescan(lo_s, lo_s + 2)

        def tbody(mm, _):
            mm_v = jnp.zeros((16,), jnp.int32) + mm
            r_v = plsc.load_gather(lidx_r, [mm_v]) + (wid * 2 - a0)
            for q in range(DIM // 16):
                vec = plsc.load_gather(tailv_r, [r_v, iota16 + q * 16])
                plsc.store_scatter(stag_r, [mm_v, iota16 + q * 16], vec)
            return 0

        lax.fori_loop(0, cnt3, tbody, 0)
        pltpu.async_copy(stag, out_hbm.at[glist], wsem).wait()

    return k


def kernel(xs, weight):
    B = xs.shape[0] * xs.shape[1]
    V = weight.shape[0]
    nfull = V // BLK
    idx = xs.astype(jnp.int32).reshape(B)
    wt = weight.T
    wtail = weight[nfull * BLK:]
    outp = _make_kernel(B, V)(idx, wt, wtail)
    return outp[:B, :DIM].reshape(xs.shape[0], xs.shape[1], DIM)


# speedup vs baseline: 9.7598x; 9.7598x over previous
import functools

import jax
import jax.numpy as jnp
from jax import lax
from jax.experimental import pallas as pl
from jax.experimental.pallas import tpu as pltpu
from jax.experimental.pallas import tpu_sc as plsc

DIM = 64
NC = 2
NS = 16
NW = NC * NS
CHUNK = 800
N_CHUNKS = 8


@functools.cache
def _make_gather(B):
    b_per_w = B // NW
    assert b_per_w == CHUNK * N_CHUNKS
    mesh = plsc.VectorSubcoreMesh(core_axis_name="c", subcore_axis_name="s")

    @functools.partial(
        pl.kernel,
        mesh=mesh,
        compiler_params=pltpu.CompilerParams(use_tc_tiling_on_sc=False),
        out_type=jax.ShapeDtypeStruct((B, 2 * DIM), jnp.float32),
        scratch_types=[
            pltpu.VMEM((N_CHUNKS, CHUNK), jnp.int32),
            pltpu.VMEM((CHUNK, DIM), jnp.float32),
            pltpu.VMEM((CHUNK, DIM), jnp.float32),
            pltpu.SemaphoreType.DMA,
            pltpu.SemaphoreType.DMA,
            pltpu.SemaphoreType.DMA,
        ],
    )
    def k(idx_hbm, table_hbm, out_hbm, idx_v, rows_a, rows_b, gsem_a, gsem_b, wsem):
        wid = lax.axis_index("s") * NC + lax.axis_index("c")
        base = wid * b_per_w
        pltpu.sync_copy(idx_hbm.at[wid], idx_v)
        bufs = (rows_a, rows_b)
        gsems = (gsem_a, gsem_b)

        def start_gather(j, slot):
            return pltpu.async_copy(table_hbm.at[idx_v.at[j]], bufs[slot], gsems[slot])

        def start_write(j, slot):
            g0 = base + j * CHUNK
            return pltpu.async_copy(
                bufs[slot], out_hbm.at[pl.ds(g0, CHUNK), pl.ds(0, DIM)], wsem)

        gathers = [None] * N_CHUNKS
        writes = [None] * N_CHUNKS
        gathers[0] = start_gather(0, 0)
        for j in range(N_CHUNKS):
            slot = j % 2
            gathers[j].wait()
            if j + 1 < N_CHUNKS:
                if j >= 1:
                    writes[j - 1].wait()
                gathers[j + 1] = start_gather(j + 1, 1 - slot)
            writes[j] = start_write(j, slot)
        writes[N_CHUNKS - 2].wait()
        writes[N_CHUNKS - 1].wait()

    return k


def kernel(xs, weight):
    B = xs.shape[0] * xs.shape[1]
    idx = xs.astype(jnp.int32).reshape(NW, N_CHUNKS, CHUNK)
    outp = _make_gather(B)(idx, weight)
    return outp[:, :DIM].reshape(xs.shape[0], xs.shape[1], DIM)


# speedup vs baseline: 10.0102x; 1.0257x over previous
import functools

import jax
import jax.numpy as jnp
from jax import lax
from jax.experimental import pallas as pl
from jax.experimental.pallas import tpu as pltpu
from jax.experimental.pallas import tpu_sc as plsc

DIM = 64
NC = 2
NS = 16
NW = NC * NS
SEQ_CHUNK = 16
N_CHUNKS = 8


@functools.cache
def _make_gather(B4, S):
    seq_per_w = B4 // NW
    chunk = SEQ_CHUNK * S
    assert seq_per_w == SEQ_CHUNK * N_CHUNKS
    mesh = plsc.VectorSubcoreMesh(core_axis_name="c", subcore_axis_name="s")

    @functools.partial(
        pl.kernel,
        mesh=mesh,
        compiler_params=pltpu.CompilerParams(use_tc_tiling_on_sc=False),
        out_type=jax.ShapeDtypeStruct((B4, S, DIM), jnp.float32),
        scratch_types=[
            pltpu.VMEM((N_CHUNKS, chunk), jnp.int32),
            pltpu.VMEM((chunk, DIM), jnp.float32),
            pltpu.VMEM((chunk, DIM), jnp.float32),
            pltpu.SemaphoreType.DMA,
            pltpu.SemaphoreType.DMA,
            pltpu.SemaphoreType.DMA,
        ],
    )
    def k(idx_hbm, table_hbm, out_hbm, idx_v, rows_a, rows_b, gsem_a, gsem_b, wsem):
        wid = lax.axis_index("s") * NC + lax.axis_index("c")
        base = wid * seq_per_w
        pltpu.sync_copy(idx_hbm.at[wid], idx_v)
        bufs = (rows_a, rows_b)
        gsems = (gsem_a, gsem_b)

        def start_gather(j, slot):
            return pltpu.async_copy(table_hbm.at[idx_v.at[j]], bufs[slot], gsems[slot])

        def start_writes(j, slot):
            b0 = base + j * SEQ_CHUNK
            return [
                pltpu.async_copy(bufs[slot].at[pl.ds(i * S, S)], out_hbm.at[b0 + i], wsem)
                for i in range(SEQ_CHUNK)
            ]

        gathers = [None] * N_CHUNKS
        writes = [None] * N_CHUNKS
        gathers[0] = start_gather(0, 0)
        for j in range(N_CHUNKS):
            slot = j % 2
            gathers[j].wait()
            if j + 1 < N_CHUNKS:
                if j >= 1:
                    for cp in writes[j - 1]:
                        cp.wait()
                gathers[j + 1] = start_gather(j + 1, 1 - slot)
            writes[j] = start_writes(j, slot)
        for cp in writes[N_CHUNKS - 2]:
            cp.wait()
        for cp in writes[N_CHUNKS - 1]:
            cp.wait()

    return k


def kernel(xs, weight):
    idx = xs.astype(jnp.int32).reshape(NW, N_CHUNKS, SEQ_CHUNK * xs.shape[1])
    return _make_gather(xs.shape[0], xs.shape[1])(idx, weight)
